# baseline (speedup 1.0000x reference)
import jax
import jax.numpy as jnp
from jax.experimental import pallas as pl


def _gat_conv(x, edge_index, W, a_src, a_dst, b):
    n = x.shape[0]
    loop = jnp.arange(n, dtype=edge_index.dtype)
    src = jnp.concatenate([edge_index[0], loop])
    dst = jnp.concatenate([edge_index[1], loop])
    h = x @ W
    alpha_src = jnp.sum(h * a_src, axis=-1)
    alpha_dst = jnp.sum(h * a_dst, axis=-1)
    e = alpha_src[src] + alpha_dst[dst]
    e = jnp.where(e > 0, e, 0.2 * e)
    e_max = jax.ops.segment_max(e, dst, num_segments=n)
    e_exp = jnp.exp(e - e_max[dst])
    denom = jax.ops.segment_sum(e_exp, dst, num_segments=n)
    alpha = e_exp / (denom[dst] + 1e-16)
    out = jax.ops.segment_sum(h[src] * alpha[:, None], dst, num_segments=n)
    return out + b


def _identity_pallas(x):
    def body(x_ref, o_ref):
        o_ref[...] = x_ref[...]
    return pl.pallas_call(body, out_shape=jax.ShapeDtypeStruct(x.shape, x.dtype))(x)


def kernel(x, edge_index, W1, a1_src, a1_dst, b1, W2, a2_src, a2_dst, b2):
    h = _gat_conv(x, edge_index, W1, a1_src, a1_dst, b1)
    h = jax.nn.relu(h)
    h = _gat_conv(h, edge_index, W2, a2_src, a2_dst, b2)
    return _identity_pallas(jax.nn.log_softmax(h, axis=1))


# speedup vs baseline: 50.1233x; 50.1233x over previous
import functools

import jax
import jax.numpy as jnp
from jax import lax
from jax.experimental import pallas as pl
from jax.experimental.pallas import tpu as pltpu
from jax.experimental.pallas import tpu_sc as plsc

N = 10000
E = 320000
D = 16
NCLS = 7
NC, NS = 2, 16
NW = NC * NS
EPW = E // NW
C = 400
G = C // 16


def _dense1_body(x_ref, W_ref, as_ref, ad_ref, h_ref, s_ref, d_ref):
    h = jnp.dot(x_ref[...], W_ref[...], preferred_element_type=jnp.float32)
    h_ref[...] = h
    s_ref[...] = jnp.dot(h, as_ref[...], preferred_element_type=jnp.float32)
    d_ref[...] = jnp.dot(h, ad_ref[...], preferred_element_type=jnp.float32)


def _combine1_body(acc_ref, den_ref, h_ref, s_ref, d_ref, b_ref, W2_ref,
                   a2s_ref, a2d_ref, h2_ref, s2_ref, d2_ref):
    e = s_ref[...] + d_ref[...]
    w = jnp.exp(jnp.where(e > 0, e, 0.2 * e))
    num = acc_ref[0] + acc_ref[1] + w * h_ref[...]
    den = den_ref[0] + den_ref[1] + w + 1e-16
    h1 = jax.nn.relu(num / den + b_ref[...])
    h2 = jnp.dot(h1, W2_ref[...], preferred_element_type=jnp.float32)
    h2_ref[...] = h2
    s2_ref[...] = jnp.dot(h2, a2s_ref[...], preferred_element_type=jnp.float32)
    d2_ref[...] = jnp.dot(h2, a2d_ref[...], preferred_element_type=jnp.float32)


def _combine2_body(acc_ref, den_ref, h2_ref, s2_ref, d2_ref, b2_ref, out_ref):
    e = s2_ref[...] + d2_ref[...]
    w = jnp.exp(jnp.where(e > 0, e, 0.2 * e))
    num = acc_ref[0] + acc_ref[1] + w * h2_ref[...]
    den = den_ref[0] + den_ref[1] + w + 1e-16
    o = num / den + b2_ref[...]
    col = lax.broadcasted_iota(jnp.int32, (1, D), 1)
    om = jnp.where(col < NCLS, o, -1e30)
    mx = jnp.max(om, axis=1, keepdims=True)
    lse = jnp.log(jnp.sum(jnp.exp(om - mx), axis=1, keepdims=True))
    res = o - mx - lse
    out_ref[...] = res[:, :NCLS]


def _edge_body(src_hbm, dst_hbm, tab_hbm, as_hbm, ad_hbm, z16_hbm, z1_hbm,
               acc_out, den_out,
               as_t, ad_t, src_b, dst_b, w_b, rows, acc_sh, den_sh, sem):
    cid = lax.axis_index("c")
    sid = lax.axis_index("s")
    wid = cid * NS + sid

    pltpu.sync_copy(as_hbm, as_t)
    pltpu.sync_copy(ad_hbm, ad_t)

    @pl.when(sid == 0)
    def _():
        pltpu.sync_copy(z16_hbm, acc_sh)
        pltpu.sync_copy(z1_hbm, den_sh)

    plsc.subcore_barrier()
    base = wid * EPW

    @pl.loop(0, EPW // C)
    def _chunk(k):
        off = base + k * C
        pltpu.sync_copy(src_hbm.at[pl.ds(off, C)], src_b)
        pltpu.sync_copy(dst_hbm.at[pl.ds(off, C)], dst_b)
        gat = pltpu.async_copy(tab_hbm.at[src_b], rows, sem)
        for g in range(G):
            sv = src_b[pl.ds(g * 16, 16)]
            dv = dst_b[pl.ds(g * 16, 16)]
            ev = plsc.load_gather(as_t, [sv]) + plsc.load_gather(ad_t, [dv])
            ev = jnp.where(ev > 0, ev, 0.2 * ev)
            w_b[pl.ds(g * 16, 16)] = jnp.exp(ev)
        gat.wait()
        for eidx in range(C):
            wb = plsc.load_gather(w_b, [jnp.full((16,), eidx, jnp.int32)])
            rows[eidx] = rows[eidx] * wb
        pltpu.sync_copy(rows, acc_sh.at[dst_b], add=True)
        pltpu.sync_copy(w_b, den_sh.at[dst_b], add=True)

    plsc.subcore_barrier()

    @pl.when(sid < 10)
    def _():
        pltpu.sync_copy(acc_sh.at[pl.ds(sid * 1000, 1000)],
                        acc_out.at[pl.ds(cid * N + sid * 1000, 1000)])

    @pl.when(sid == 0)
    def _():
        pltpu.sync_copy(den_sh, den_out.at[pl.ds(cid * N, N)])


_edge_pass = functools.partial(
    pl.kernel,
    out_type=(jax.ShapeDtypeStruct((NC * N, D), jnp.float32),
              jax.ShapeDtypeStruct((NC * N,), jnp.float32)),
    mesh=plsc.VectorSubcoreMesh(core_axis_name="c", subcore_axis_name="s",
                                num_cores=NC, num_subcores=NS),
    compiler_params=pltpu.CompilerParams(needs_layout_passes=False,
                                         use_tc_tiling_on_sc=False),
    scratch_types=[
        pltpu.VMEM((N,), jnp.float32),
        pltpu.VMEM((N,), jnp.float32),
        pltpu.VMEM((C,), jnp.int32),
        pltpu.VMEM((C,), jnp.int32),
        pltpu.VMEM((C,), jnp.float32),
        pltpu.VMEM((C, D), jnp.float32),
        pltpu.VMEM_SHARED((N, D), jnp.float32),
        pltpu.VMEM_SHARED((N,), jnp.float32),
        pltpu.SemaphoreType.DMA,
    ],
)(_edge_body)


_dense1 = pl.pallas_call(
    _dense1_body,
    out_shape=(jax.ShapeDtypeStruct((N, D), jnp.float32),
               jax.ShapeDtypeStruct((N, 1), jnp.float32),
               jax.ShapeDtypeStruct((N, 1), jnp.float32)),
)

_combine1 = pl.pallas_call(
    _combine1_body,
    out_shape=(jax.ShapeDtypeStruct((N, D), jnp.float32),
               jax.ShapeDtypeStruct((N, 1), jnp.float32),
               jax.ShapeDtypeStruct((N, 1), jnp.float32)),
)

_combine2 = pl.pallas_call(
    _combine2_body,
    out_shape=jax.ShapeDtypeStruct((N, NCLS), jnp.float32),
)


def kernel(x, edge_index, W1, a1_src, a1_dst, b1, W2, a2_src, a2_dst, b2):
    src = edge_index[0]
    dst = edge_index[1]

    h1, s1, d1 = _dense1(x, W1, a1_src.reshape(D, 1), a1_dst.reshape(D, 1))

    z16 = jnp.zeros((N, D), jnp.float32)
    z1 = jnp.zeros((N,), jnp.float32)
    acc1, den1 = _edge_pass(src, dst, h1, s1.reshape(N), d1.reshape(N), z16, z1)

    W2p = jnp.zeros((D, D), jnp.float32).at[:, :NCLS].set(W2)
    a2sp = jnp.zeros((D, 1), jnp.float32).at[:NCLS, 0].set(a2_src)
    a2dp = jnp.zeros((D, 1), jnp.float32).at[:NCLS, 0].set(a2_dst)
    b2p = jnp.zeros((1, D), jnp.float32).at[0, :NCLS].set(b2)

    h2, s2, d2 = _combine1(acc1.reshape(NC, N, D), den1.reshape(NC, N, 1),
                           h1, s1, d1, b1.reshape(1, D), W2p, a2sp, a2dp)

    acc2, den2 = _edge_pass(src, dst, h2, s2.reshape(N), d2.reshape(N), z16, z1)

    return _combine2(acc2.reshape(NC, N, D), den2.reshape(NC, N, 1),
                     h2, s2, d2, b2p)


# speedup vs baseline: 56.0174x; 1.1176x over previous
import functools

import jax
import jax.numpy as jnp
from jax import lax
from jax.experimental import pallas as pl
from jax.experimental.pallas import tpu as pltpu
from jax.experimental.pallas import tpu_sc as plsc

N = 10000
E = 320000
D = 16
NCLS = 7
NC, NS = 2, 16
NW = NC * NS
EPW = E // NW
C = 400
G = C // 16
NCH = EPW // C


def _dense1_body(x_ref, W_ref, as_ref, ad_ref, h_ref, s_ref, d_ref):
    h = jnp.dot(x_ref[...], W_ref[...])
    h_ref[...] = h
    s_ref[...] = jnp.sum(h * as_ref[...], axis=1, keepdims=True)
    d_ref[...] = jnp.sum(h * ad_ref[...], axis=1, keepdims=True)


def _combine1_body(acc_ref, den_ref, h_ref, s_ref, d_ref, b_ref, W2_ref,
                   a2s_ref, a2d_ref, h2_ref, s2_ref, d2_ref):
    e = s_ref[...] + d_ref[...]
    w = jnp.exp(jnp.where(e > 0, e, 0.2 * e))
    num = acc_ref[0] + acc_ref[1] + w * h_ref[...]
    den = den_ref[0] + den_ref[1] + w + 1e-16
    h1 = jax.nn.relu(num / den + b_ref[...])
    h2 = jnp.dot(h1, W2_ref[...])
    h2_ref[...] = h2
    s2_ref[...] = jnp.sum(h2 * a2s_ref[...], axis=1, keepdims=True)
    d2_ref[...] = jnp.sum(h2 * a2d_ref[...], axis=1, keepdims=True)


def _combine2_body(acc_ref, den_ref, h2_ref, s2_ref, d2_ref, b2_ref, out_ref):
    e = s2_ref[...] + d2_ref[...]
    w = jnp.exp(jnp.where(e > 0, e, 0.2 * e))
    num = acc_ref[0] + acc_ref[1] + w * h2_ref[...]
    den = den_ref[0] + den_ref[1] + w + 1e-16
    o = num / den + b2_ref[...]
    col = lax.broadcasted_iota(jnp.int32, (1, D), 1)
    om = jnp.where(col < NCLS, o, -1e30)
    mx = jnp.max(om, axis=1, keepdims=True)
    lse = jnp.log(jnp.sum(jnp.exp(om - mx), axis=1, keepdims=True))
    res = o - mx - lse
    out_ref[...] = res[:, :NCLS]


def _edge_body(src_hbm, dst_hbm, tab_hbm, as_hbm, ad_hbm, z16_hbm, z1_hbm,
               acc_out, den_out,
               as_t, ad_t, src_b, dst_b, w_b, rows, acc_sh, den_sh,
               sem_s, sem_d, sem_g):
    cid = lax.axis_index("c")
    sid = lax.axis_index("s")
    wid = cid * NS + sid

    pltpu.sync_copy(as_hbm, as_t)
    pltpu.sync_copy(ad_hbm, ad_t)

    @pl.when(sid == 0)
    def _():
        pltpu.sync_copy(z16_hbm, acc_sh)
        pltpu.sync_copy(z1_hbm, den_sh)

    plsc.subcore_barrier()
    base = wid * EPW

    def start_idx(k):
        b = k % 2
        off = base + k * C
        pltpu.async_copy(src_hbm.at[pl.ds(off, C)], src_b.at[b], sem_s.at[b])
        pltpu.async_copy(dst_hbm.at[pl.ds(off, C)], dst_b.at[b], sem_d.at[b])

    def wait_idx(b):
        pltpu.make_async_copy(src_hbm.at[pl.ds(0, C)], src_b.at[b],
                              sem_s.at[b]).wait()
        pltpu.make_async_copy(dst_hbm.at[pl.ds(0, C)], dst_b.at[b],
                              sem_d.at[b]).wait()

    def start_gather(b):
        pltpu.async_copy(tab_hbm.at[src_b.at[b]], rows.at[b], sem_g.at[b])

    def wait_gather(b):
        pltpu.make_async_copy(tab_hbm.at[src_b.at[b]], rows.at[b],
                              sem_g.at[b]).wait()

    start_idx(0)
    start_idx(1)
    wait_idx(0)
    start_gather(0)

    @pl.loop(0, NCH)
    def _chunk(k):
        b = k % 2
        for g in range(G):
            sv = src_b[b, pl.ds(g * 16, 16)]
            dv = dst_b[b, pl.ds(g * 16, 16)]
            ev = plsc.load_gather(as_t, [sv]) + plsc.load_gather(ad_t, [dv])
            ev = jnp.where(ev > 0, ev, 0.2 * ev)
            w_b[pl.ds(g * 16, 16)] = jnp.exp(ev)
        wait_gather(b)

        for eidx in range(C):
            wb = plsc.load_gather(w_b, [jnp.full((16,), eidx, jnp.int32)])
            rows[b, eidx] = rows[b, eidx] * wb

        @pl.when(k + 1 < NCH)
        def _():
            wait_idx(1 - b)
            start_gather(1 - b)

        pltpu.sync_copy(rows.at[b], acc_sh.at[dst_b.at[b]], add=True)
        pltpu.sync_copy(w_b, den_sh.at[dst_b.at[b]], add=True)

        @pl.when(k + 2 < NCH)
        def _():
            start_idx(k + 2)

    plsc.subcore_barrier()

    @pl.when(sid < 10)
    def _():
        pltpu.sync_copy(acc_sh.at[pl.ds(sid * 1000, 1000)],
                        acc_out.at[pl.ds(cid * N + sid * 1000, 1000)])

    @pl.when(sid == 0)
    def _():
        pltpu.sync_copy(den_sh, den_out.at[pl.ds(cid * N, N)])


_edge_pass = functools.partial(
    pl.kernel,
    out_type=(jax.ShapeDtypeStruct((NC * N, D), jnp.float32),
              jax.ShapeDtypeStruct((NC * N,), jnp.float32)),
    mesh=plsc.VectorSubcoreMesh(core_axis_name="c", subcore_axis_name="s",
                                num_cores=NC, num_subcores=NS),
    compiler_params=pltpu.CompilerParams(needs_layout_passes=False,
                                         use_tc_tiling_on_sc=False),
    scratch_types=[
        pltpu.VMEM((N,), jnp.float32),
        pltpu.VMEM((N,), jnp.float32),
        pltpu.VMEM((2, C), jnp.int32),
        pltpu.VMEM((2, C), jnp.int32),
        pltpu.VMEM((C,), jnp.float32),
        pltpu.VMEM((2, C, D), jnp.float32),
        pltpu.VMEM_SHARED((N, D), jnp.float32),
        pltpu.VMEM_SHARED((N,), jnp.float32),
        pltpu.SemaphoreType.DMA((2,)),
        pltpu.SemaphoreType.DMA((2,)),
        pltpu.SemaphoreType.DMA((2,)),
    ],
)(_edge_body)


_dense1 = pl.pallas_call(
    _dense1_body,
    out_shape=(jax.ShapeDtypeStruct((N, D), jnp.float32),
               jax.ShapeDtypeStruct((N, 1), jnp.float32),
               jax.ShapeDtypeStruct((N, 1), jnp.float32)),
)

_combine1 = pl.pallas_call(
    _combine1_body,
    out_shape=(jax.ShapeDtypeStruct((N, D), jnp.float32),
               jax.ShapeDtypeStruct((N, 1), jnp.float32),
               jax.ShapeDtypeStruct((N, 1), jnp.float32)),
)

_combine2 = pl.pallas_call(
    _combine2_body,
    out_shape=jax.ShapeDtypeStruct((N, NCLS), jnp.float32),
)


def kernel(x, edge_index, W1, a1_src, a1_dst, b1, W2, a2_src, a2_dst, b2):
    src = edge_index[0]
    dst = edge_index[1]

    h1, s1, d1 = _dense1(x, W1, a1_src.reshape(1, D), a1_dst.reshape(1, D))

    z16 = jnp.zeros((N, D), jnp.float32)
    z1 = jnp.zeros((N,), jnp.float32)
    acc1, den1 = _edge_pass(src, dst, h1, s1.reshape(N), d1.reshape(N), z16, z1)

    W2p = jnp.zeros((D, D), jnp.float32).at[:, :NCLS].set(W2)
    a2sp = jnp.zeros((1, D), jnp.float32).at[0, :NCLS].set(a2_src)
    a2dp = jnp.zeros((1, D), jnp.float32).at[0, :NCLS].set(a2_dst)
    b2p = jnp.zeros((1, D), jnp.float32).at[0, :NCLS].set(b2)

    h2, s2, d2 = _combine1(acc1.reshape(NC, N, D), den1.reshape(NC, N, 1),
                           h1, s1, d1, b1.reshape(1, D), W2p, a2sp, a2dp)

    acc2, den2 = _edge_pass(src, dst, h2, s2.reshape(N), d2.reshape(N), z16, z1)

    return _combine2(acc2.reshape(NC, N, D), den2.reshape(NC, N, 1),
                     h2, s2, d2, b2p)


# speedup vs baseline: 56.1539x; 1.0024x over previous
import functools

import jax
import jax.numpy as jnp
from jax import lax
from jax.experimental import pallas as pl
from jax.experimental.pallas import tpu as pltpu
from jax.experimental.pallas import tpu_sc as plsc

N = 10000
E = 320000
D = 16
NCLS = 7
NC, NS = 2, 16
NW = NC * NS
EPW = E // NW
C = 400
G = C // 16
NCH = EPW // C


def _dense1_body(x_ref, W_ref, as_ref, ad_ref, h_ref, s_ref, d_ref):
    h = jnp.dot(x_ref[...], W_ref[...])
    h_ref[...] = h
    s_ref[...] = jnp.sum(h * as_ref[...], axis=1, keepdims=True)
    d_ref[...] = jnp.sum(h * ad_ref[...], axis=1, keepdims=True)


def _combine1_body(acc_ref, den_ref, h_ref, s_ref, d_ref, b_ref, W2_ref,
                   a2s_ref, a2d_ref, h2_ref, s2_ref, d2_ref):
    e = s_ref[...] + d_ref[...]
    w = jnp.exp(jnp.where(e > 0, e, 0.2 * e))
    num = acc_ref[0] + acc_ref[1] + w * h_ref[...]
    den = den_ref[0] + den_ref[1] + w + 1e-16
    h1 = jax.nn.relu(num / den + b_ref[...])
    h2 = jnp.dot(h1, W2_ref[...])
    h2_ref[...] = h2
    s2_ref[...] = jnp.sum(h2 * a2s_ref[...], axis=1, keepdims=True)
    d2_ref[...] = jnp.sum(h2 * a2d_ref[...], axis=1, keepdims=True)


def _combine2_body(acc_ref, den_ref, h2_ref, s2_ref, d2_ref, b2_ref, out_ref):
    e = s2_ref[...] + d2_ref[...]
    w = jnp.exp(jnp.where(e > 0, e, 0.2 * e))
    num = acc_ref[0] + acc_ref[1] + w * h2_ref[...]
    den = den_ref[0] + den_ref[1] + w + 1e-16
    o = num / den + b2_ref[...]
    col = lax.broadcasted_iota(jnp.int32, (1, D), 1)
    om = jnp.where(col < NCLS, o, -1e30)
    mx = jnp.max(om, axis=1, keepdims=True)
    lse = jnp.log(jnp.sum(jnp.exp(om - mx), axis=1, keepdims=True))
    res = o - mx - lse
    out_ref[...] = res[:, :NCLS]


def _edge_body(src_hbm, dst_hbm, tab_hbm, as_hbm, ad_hbm, z16_hbm, z1_hbm,
               acc_out, den_out,
               as_t, ad_t, src_b, dst_b, w_b, rows, acc_sh, den_sh,
               sem_s, sem_d, sem_g, sem_a, sem_w):
    cid = lax.axis_index("c")
    sid = lax.axis_index("s")
    wid = cid * NS + sid

    pltpu.sync_copy(as_hbm, as_t)
    pltpu.sync_copy(ad_hbm, ad_t)

    @pl.when(sid == 0)
    def _():
        pltpu.sync_copy(z16_hbm, acc_sh)
        pltpu.sync_copy(z1_hbm, den_sh)

    plsc.subcore_barrier()
    base = wid * EPW

    def start_idx(k):
        b = k % 2
        off = base + k * C
        pltpu.async_copy(src_hbm.at[pl.ds(off, C)], src_b.at[b], sem_s.at[b])
        pltpu.async_copy(dst_hbm.at[pl.ds(off, C)], dst_b.at[b], sem_d.at[b])

    def wait_idx(b):
        pltpu.make_async_copy(src_hbm.at[pl.ds(0, C)], src_b.at[b],
                              sem_s.at[b]).wait()
        pltpu.make_async_copy(dst_hbm.at[pl.ds(0, C)], dst_b.at[b],
                              sem_d.at[b]).wait()

    def start_gather(b):
        pltpu.async_copy(tab_hbm.at[src_b.at[b]], rows.at[b], sem_g.at[b])

    def wait_gather(b):
        pltpu.make_async_copy(tab_hbm.at[src_b.at[b]], rows.at[b],
                              sem_g.at[b]).wait()

    def start_scatter(b):
        pltpu.async_copy(rows.at[b], acc_sh.at[dst_b.at[b]], sem_a.at[b],
                         add=True)
        pltpu.async_copy(w_b.at[b], den_sh.at[dst_b.at[b]], sem_w.at[b],
                         add=True)

    def drain_scatter(b):
        pltpu.make_async_copy(rows.at[b], acc_sh.at[dst_b.at[b]],
                              sem_a.at[b]).wait()
        pltpu.make_async_copy(w_b.at[b], den_sh.at[dst_b.at[b]],
                              sem_w.at[b]).wait()

    start_idx(0)
    wait_idx(0)
    start_gather(0)

    @pl.loop(0, NCH)
    def _chunk(k):
        b = k % 2

        @pl.when(k > 0)
        def _():
            drain_scatter(1 - b)

        @pl.when(k + 1 < NCH)
        def _():
            start_idx(k + 1)

        for g in range(G):
            sv = src_b[b, pl.ds(g * 16, 16)]
            dv = dst_b[b, pl.ds(g * 16, 16)]
            ev = plsc.load_gather(as_t, [sv]) + plsc.load_gather(ad_t, [dv])
            ev = jnp.where(ev > 0, ev, 0.2 * ev)
            w_b[b, pl.ds(g * 16, 16)] = jnp.exp(ev)
        wait_gather(b)

        for eidx in range(C):
            wb = plsc.load_gather(w_b.at[b], [jnp.full((16,), eidx, jnp.int32)])
            rows[b, eidx] = rows[b, eidx] * wb

        @pl.when(k + 1 < NCH)
        def _():
            wait_idx(1 - b)
            start_gather(1 - b)

        start_scatter(b)

    drain_scatter((NCH - 1) % 2)
    plsc.subcore_barrier()

    @pl.when(sid < 10)
    def _():
        pltpu.sync_copy(acc_sh.at[pl.ds(sid * 1000, 1000)],
                        acc_out.at[pl.ds(cid * N + sid * 1000, 1000)])

    @pl.when(sid == 0)
    def _():
        pltpu.sync_copy(den_sh, den_out.at[pl.ds(cid * N, N)])


_edge_pass = functools.partial(
    pl.kernel,
    out_type=(jax.ShapeDtypeStruct((NC * N, D), jnp.float32),
              jax.ShapeDtypeStruct((NC * N,), jnp.float32)),
    mesh=plsc.VectorSubcoreMesh(core_axis_name="c", subcore_axis_name="s",
                                num_cores=NC, num_subcores=NS),
    compiler_params=pltpu.CompilerParams(needs_layout_passes=False,
                                         use_tc_tiling_on_sc=False),
    scratch_types=[
        pltpu.VMEM((N,), jnp.float32),
        pltpu.VMEM((N,), jnp.float32),
        pltpu.VMEM((2, C), jnp.int32),
        pltpu.VMEM((2, C), jnp.int32),
        pltpu.VMEM((2, C), jnp.float32),
        pltpu.VMEM((2, C, D), jnp.float32),
        pltpu.VMEM_SHARED((N, D), jnp.float32),
        pltpu.VMEM_SHARED((N,), jnp.float32),
        pltpu.SemaphoreType.DMA((2,)),
        pltpu.SemaphoreType.DMA((2,)),
        pltpu.SemaphoreType.DMA((2,)),
        pltpu.SemaphoreType.DMA((2,)),
        pltpu.SemaphoreType.DMA((2,)),
    ],
)(_edge_body)


_dense1 = pl.pallas_call(
    _dense1_body,
    out_shape=(jax.ShapeDtypeStruct((N, D), jnp.float32),
               jax.ShapeDtypeStruct((N, 1), jnp.float32),
               jax.ShapeDtypeStruct((N, 1), jnp.float32)),
)

_combine1 = pl.pallas_call(
    _combine1_body,
    out_shape=(jax.ShapeDtypeStruct((N, D), jnp.float32),
               jax.ShapeDtypeStruct((N, 1), jnp.float32),
               jax.ShapeDtypeStruct((N, 1), jnp.float32)),
)

_combine2 = pl.pallas_call(
    _combine2_body,
    out_shape=jax.ShapeDtypeStruct((N, NCLS), jnp.float32),
)


def kernel(x, edge_index, W1, a1_src, a1_dst, b1, W2, a2_src, a2_dst, b2):
    src = edge_index[0]
    dst = edge_index[1]

    h1, s1, d1 = _dense1(x, W1, a1_src.reshape(1, D), a1_dst.reshape(1, D))

    z16 = jnp.zeros((N, D), jnp.float32)
    z1 = jnp.zeros((N,), jnp.float32)
    acc1, den1 = _edge_pass(src, dst, h1, s1.reshape(N), d1.reshape(N), z16, z1)

    W2p = jnp.zeros((D, D), jnp.float32).at[:, :NCLS].set(W2)
    a2sp = jnp.zeros((1, D), jnp.float32).at[0, :NCLS].set(a2_src)
    a2dp = jnp.zeros((1, D), jnp.float32).at[0, :NCLS].set(a2_dst)
    b2p = jnp.zeros((1, D), jnp.float32).at[0, :NCLS].set(b2)

    h2, s2, d2 = _combine1(acc1.reshape(NC, N, D), den1.reshape(NC, N, 1),
                           h1, s1, d1, b1.reshape(1, D), W2p, a2sp, a2dp)

    acc2, den2 = _edge_pass(src, dst, h2, s2.reshape(N), d2.reshape(N), z16, z1)

    return _combine2(acc2.reshape(NC, N, D), den2.reshape(NC, N, 1),
                     h2, s2, d2, b2p)


# speedup vs baseline: 85.1572x; 1.5165x over previous
import functools

import jax
import jax.numpy as jnp
from jax import lax
from jax.experimental import pallas as pl
from jax.experimental.pallas import tpu as pltpu
from jax.experimental.pallas import tpu_sc as plsc

N = 10000
E = 320000
D = 16
NCLS = 7
NC, NS = 2, 16
NW = NC * NS
EPW = E // NW
C = 400
G = C // 16
NCH = EPW // C


def _dense1_body(x_ref, W_ref, as_ref, ad_ref, h_ref, s_ref, d_ref):
    h = jnp.dot(x_ref[...], W_ref[...])
    h_ref[...] = h
    s_ref[...] = jnp.sum(h * as_ref[...], axis=1, keepdims=True)
    d_ref[...] = jnp.sum(h * ad_ref[...], axis=1, keepdims=True)


def _combine1_body(acc_ref, den_ref, h_ref, s_ref, d_ref, b_ref, W2_ref,
                   a2s_ref, a2d_ref, h2_ref, s2_ref, d2_ref):
    e = s_ref[...] + d_ref[...]
    w = jnp.exp(jnp.where(e > 0, e, 0.2 * e))
    num = acc_ref[0] + acc_ref[1] + w * h_ref[...]
    den = den_ref[0] + den_ref[1] + w + 1e-16
    h1 = jax.nn.relu(num / den + b_ref[...])
    h2 = jnp.dot(h1, W2_ref[...])
    h2_ref[...] = h2
    s2_ref[...] = jnp.sum(h2 * a2s_ref[...], axis=1, keepdims=True)
    d2_ref[...] = jnp.sum(h2 * a2d_ref[...], axis=1, keepdims=True)


def _combine2_body(acc_ref, den_ref, h2_ref, s2_ref, d2_ref, b2_ref, out_ref):
    e = s2_ref[...] + d2_ref[...]
    w = jnp.exp(jnp.where(e > 0, e, 0.2 * e))
    num = acc_ref[0] + acc_ref[1] + w * h2_ref[...]
    den = den_ref[0] + den_ref[1] + w + 1e-16
    o = num / den + b2_ref[...]
    col = lax.broadcasted_iota(jnp.int32, (1, D), 1)
    om = jnp.where(col < NCLS, o, -1e30)
    mx = jnp.max(om, axis=1, keepdims=True)
    lse = jnp.log(jnp.sum(jnp.exp(om - mx), axis=1, keepdims=True))
    res = o - mx - lse
    out_ref[...] = res[:, :NCLS]


def _edge_body(src_hbm, dst_hbm, tab_hbm, as_hbm, ad_hbm, z16_hbm, z1_hbm,
               acc_out, den_out,
               as_t, ad_t, src_b, dst_b, w_b, rows, acc_sh, den_sh,
               sem_s, sem_d, sem_g, sem_a, sem_w):
    cid = lax.axis_index("c")
    sid = lax.axis_index("s")
    wid = cid * NS + sid

    pltpu.sync_copy(as_hbm, as_t)
    pltpu.sync_copy(ad_hbm, ad_t)

    @pl.when(sid == 0)
    def _():
        pltpu.sync_copy(z16_hbm, acc_sh)
        pltpu.sync_copy(z1_hbm, den_sh)

    plsc.subcore_barrier()
    base = wid * EPW

    def start_idx(k):
        b = k % 2
        off = base + k * C
        pltpu.async_copy(src_hbm.at[pl.ds(off, C)], src_b.at[b], sem_s.at[b])
        pltpu.async_copy(dst_hbm.at[pl.ds(off, C)], dst_b.at[b], sem_d.at[b])

    def wait_idx(b):
        pltpu.make_async_copy(src_hbm.at[pl.ds(0, C)], src_b.at[b],
                              sem_s.at[b]).wait()
        pltpu.make_async_copy(dst_hbm.at[pl.ds(0, C)], dst_b.at[b],
                              sem_d.at[b]).wait()

    def start_gather(b):
        pltpu.async_copy(tab_hbm.at[src_b.at[b]], rows.at[b], sem_g.at[b])

    def wait_gather(b):
        pltpu.make_async_copy(tab_hbm.at[src_b.at[b]], rows.at[b],
                              sem_g.at[b]).wait()

    def start_scatter(b):
        pltpu.async_copy(rows.at[b], acc_sh.at[dst_b.at[b]], sem_a.at[b],
                         add=True)
        pltpu.async_copy(w_b.at[b], den_sh.at[dst_b.at[b]], sem_w.at[b],
                         add=True)

    def drain_scatter(b):
        pltpu.make_async_copy(rows.at[b], acc_sh.at[dst_b.at[b]],
                              sem_a.at[b]).wait()
        pltpu.make_async_copy(w_b.at[b], den_sh.at[dst_b.at[b]],
                              sem_w.at[b]).wait()

    start_idx(0)
    wait_idx(0)
    start_gather(0)

    @pl.loop(0, NCH)
    def _chunk(k):
        b = k % 2

        @pl.when(k > 0)
        def _():
            drain_scatter(1 - b)

        @pl.when(k + 1 < NCH)
        def _():
            start_idx(k + 1)

        for g in range(G):
            sv = src_b[b, pl.ds(g * 16, 16)]
            dv = dst_b[b, pl.ds(g * 16, 16)]
            ev = plsc.load_gather(as_t, [sv]) + plsc.load_gather(ad_t, [dv])
            ev = jnp.where(ev > 0, ev, 0.2 * ev)
            w_b[b, pl.ds(g * 16, 16)] = jnp.exp(ev)
        wait_gather(b)

        for g in range(G):
            wv = w_b[b, pl.ds(g * 16, 16)]
            for j in range(16):
                wb = jnp.broadcast_to(wv[j], (16,))
                e = g * 16 + j
                rows[b, e] = rows[b, e] * wb

        @pl.when(k + 1 < NCH)
        def _():
            wait_idx(1 - b)
            start_gather(1 - b)

        start_scatter(b)

    drain_scatter((NCH - 1) % 2)
    plsc.subcore_barrier()

    @pl.when(sid < 10)
    def _():
        pltpu.sync_copy(acc_sh.at[pl.ds(sid * 1000, 1000)],
                        acc_out.at[pl.ds(cid * N + sid * 1000, 1000)])

    @pl.when(sid == 0)
    def _():
        pltpu.sync_copy(den_sh, den_out.at[pl.ds(cid * N, N)])


_edge_pass = functools.partial(
    pl.kernel,
    out_type=(jax.ShapeDtypeStruct((NC * N, D), jnp.float32),
              jax.ShapeDtypeStruct((NC * N,), jnp.float32)),
    mesh=plsc.VectorSubcoreMesh(core_axis_name="c", subcore_axis_name="s",
                                num_cores=NC, num_subcores=NS),
    compiler_params=pltpu.CompilerParams(needs_layout_passes=False,
                                         use_tc_tiling_on_sc=False),
    scratch_types=[
        pltpu.VMEM((N,), jnp.float32),
        pltpu.VMEM((N,), jnp.float32),
        pltpu.VMEM((2, C), jnp.int32),
        pltpu.VMEM((2, C), jnp.int32),
        pltpu.VMEM((2, C), jnp.float32),
        pltpu.VMEM((2, C, D), jnp.float32),
        pltpu.VMEM_SHARED((N, D), jnp.float32),
        pltpu.VMEM_SHARED((N,), jnp.float32),
        pltpu.SemaphoreType.DMA((2,)),
        pltpu.SemaphoreType.DMA((2,)),
        pltpu.SemaphoreType.DMA((2,)),
        pltpu.SemaphoreType.DMA((2,)),
        pltpu.SemaphoreType.DMA((2,)),
    ],
)(_edge_body)


_dense1 = pl.pallas_call(
    _dense1_body,
    out_shape=(jax.ShapeDtypeStruct((N, D), jnp.float32),
               jax.ShapeDtypeStruct((N, 1), jnp.float32),
               jax.ShapeDtypeStruct((N, 1), jnp.float32)),
)

_combine1 = pl.pallas_call(
    _combine1_body,
    out_shape=(jax.ShapeDtypeStruct((N, D), jnp.float32),
               jax.ShapeDtypeStruct((N, 1), jnp.float32),
               jax.ShapeDtypeStruct((N, 1), jnp.float32)),
)

_combine2 = pl.pallas_call(
    _combine2_body,
    out_shape=jax.ShapeDtypeStruct((N, NCLS), jnp.float32),
)


def kernel(x, edge_index, W1, a1_src, a1_dst, b1, W2, a2_src, a2_dst, b2):
    src = edge_index[0]
    dst = edge_index[1]

    h1, s1, d1 = _dense1(x, W1, a1_src.reshape(1, D), a1_dst.reshape(1, D))

    z16 = jnp.zeros((N, D), jnp.float32)
    z1 = jnp.zeros((N,), jnp.float32)
    acc1, den1 = _edge_pass(src, dst, h1, s1.reshape(N), d1.reshape(N), z16, z1)

    W2p = jnp.zeros((D, D), jnp.float32).at[:, :NCLS].set(W2)
    a2sp = jnp.zeros((1, D), jnp.float32).at[0, :NCLS].set(a2_src)
    a2dp = jnp.zeros((1, D), jnp.float32).at[0, :NCLS].set(a2_dst)
    b2p = jnp.zeros((1, D), jnp.float32).at[0, :NCLS].set(b2)

    h2, s2, d2 = _combine1(acc1.reshape(NC, N, D), den1.reshape(NC, N, 1),
                           h1, s1, d1, b1.reshape(1, D), W2p, a2sp, a2dp)

    acc2, den2 = _edge_pass(src, dst, h2, s2.reshape(N), d2.reshape(N), z16, z1)

    return _combine2(acc2.reshape(NC, N, D), den2.reshape(NC, N, 1),
                     h2, s2, d2, b2p)


# speedup vs baseline: 106.4008x; 1.2495x over previous
import functools

import jax
import jax.numpy as jnp
from jax import lax
from jax.experimental import pallas as pl
from jax.experimental.pallas import tpu as pltpu
from jax.experimental.pallas import tpu_sc as plsc

N = 10000
E = 320000
D = 16
NCLS = 7
NC, NS = 2, 16
NW = NC * NS
EPW = E // NW
C = 400
G = C // 16
NCH = EPW // C


def _dense1_body(x_ref, W_ref, as_ref, ad_ref, h_ref, s_ref, d_ref):
    h = jnp.dot(x_ref[...], W_ref[...])
    h_ref[...] = h
    s_ref[...] = jnp.sum(h * as_ref[...], axis=1)
    d_ref[...] = jnp.sum(h * ad_ref[...], axis=1)


def _self_weight(h, as_row, ad_row):
    e = (jnp.sum(h * as_row, axis=1, keepdims=True)
         + jnp.sum(h * ad_row, axis=1, keepdims=True))
    eb = jnp.broadcast_to(e, h.shape)
    return jnp.exp(jnp.where(eb > 0, eb, 0.2 * eb))


def _combine1_body(acc_ref, den_ref, h_ref, as_ref, ad_ref, b_ref, W2_ref,
                   a2s_ref, a2d_ref, h2_ref, s2_ref, d2_ref):
    h = h_ref[...]
    w = _self_weight(h, as_ref[...], ad_ref[...])
    num = acc_ref[pl.ds(0, N)] + acc_ref[pl.ds(N, N)] + w * h
    den = den_ref[pl.ds(0, N)] + den_ref[pl.ds(N, N)] + w[:, 0] + 1e-16
    denb = jnp.broadcast_to(den.reshape(N, 1), (N, D))
    h1 = jax.nn.relu(num / denb + b_ref[...])
    h2 = jnp.dot(h1, W2_ref[...])
    h2_ref[...] = h2
    s2_ref[...] = jnp.sum(h2 * a2s_ref[...], axis=1)
    d2_ref[...] = jnp.sum(h2 * a2d_ref[...], axis=1)


def _combine2_body(acc_ref, den_ref, h2_ref, a2s_ref, a2d_ref, b2_ref, out_ref):
    h2 = h2_ref[...]
    w = _self_weight(h2, a2s_ref[...], a2d_ref[...])
    num = acc_ref[pl.ds(0, N)] + acc_ref[pl.ds(N, N)] + w * h2
    den = den_ref[pl.ds(0, N)] + den_ref[pl.ds(N, N)] + w[:, 0] + 1e-16
    denb = jnp.broadcast_to(den.reshape(N, 1), (N, D))
    o = num / denb + b2_ref[...]
    col = lax.broadcasted_iota(jnp.int32, (1, D), 1)
    om = jnp.where(col < NCLS, o, -1e30)
    mx = jnp.max(om, axis=1, keepdims=True)
    mxb = jnp.broadcast_to(mx, (N, D))
    lse = jnp.log(jnp.broadcast_to(
        jnp.sum(jnp.exp(om - mxb), axis=1, keepdims=True), (N, D)))
    res = o - mxb - lse
    out_ref[...] = res[:, :NCLS]


def _edge_body(edge_hbm, tab_hbm, as_hbm, ad_hbm, z16_hbm, z1_hbm,
               acc_out, den_out,
               as_t, ad_t, src_b, dst_b, w_b, rows, acc_sh, den_sh,
               sem_s, sem_d, sem_g, sem_a, sem_w):
    cid = lax.axis_index("c")
    sid = lax.axis_index("s")
    wid = cid * NS + sid

    pltpu.sync_copy(as_hbm, as_t)
    pltpu.sync_copy(ad_hbm, ad_t)

    @pl.when(sid == 0)
    def _():
        pltpu.sync_copy(z16_hbm, acc_sh)
        pltpu.sync_copy(z1_hbm, den_sh)

    plsc.subcore_barrier()
    base = wid * EPW

    def start_idx(k):
        b = k % 2
        off = base + k * C
        pltpu.async_copy(edge_hbm.at[0, pl.ds(off, C)], src_b.at[b],
                         sem_s.at[b])
        pltpu.async_copy(edge_hbm.at[1, pl.ds(off, C)], dst_b.at[b],
                         sem_d.at[b])

    def wait_idx(b):
        pltpu.make_async_copy(edge_hbm.at[0, pl.ds(0, C)], src_b.at[b],
                              sem_s.at[b]).wait()
        pltpu.make_async_copy(edge_hbm.at[1, pl.ds(0, C)], dst_b.at[b],
                              sem_d.at[b]).wait()

    def start_gather(b):
        pltpu.async_copy(tab_hbm.at[src_b.at[b]], rows.at[b], sem_g.at[b])

    def wait_gather(b):
        pltpu.make_async_copy(tab_hbm.at[src_b.at[b]], rows.at[b],
                              sem_g.at[b]).wait()

    def start_scatter(b):
        pltpu.async_copy(rows.at[b], acc_sh.at[dst_b.at[b]], sem_a.at[b],
                         add=True)
        pltpu.async_copy(w_b.at[b], den_sh.at[dst_b.at[b]], sem_w.at[b],
                         add=True)

    def drain_scatter(b):
        pltpu.make_async_copy(rows.at[b], acc_sh.at[dst_b.at[b]],
                              sem_a.at[b]).wait()
        pltpu.make_async_copy(w_b.at[b], den_sh.at[dst_b.at[b]],
                              sem_w.at[b]).wait()

    start_idx(0)
    wait_idx(0)
    start_gather(0)

    @pl.loop(0, NCH)
    def _chunk(k):
        b = k % 2

        @pl.when(k > 0)
        def _():
            drain_scatter(1 - b)

        @pl.when(k + 1 < NCH)
        def _():
            start_idx(k + 1)

        for g in range(G):
            sv = src_b[b, pl.ds(g * 16, 16)]
            dv = dst_b[b, pl.ds(g * 16, 16)]
            ev = plsc.load_gather(as_t, [sv]) + plsc.load_gather(ad_t, [dv])
            ev = jnp.where(ev > 0, ev, 0.2 * ev)
            w_b[b, pl.ds(g * 16, 16)] = jnp.exp(ev)
        wait_gather(b)

        for g in range(G):
            wv = w_b[b, pl.ds(g * 16, 16)]
            for j in range(16):
                wb = jnp.broadcast_to(wv[j], (16,))
                e = g * 16 + j
                rows[b, e] = rows[b, e] * wb

        @pl.when(k + 1 < NCH)
        def _():
            wait_idx(1 - b)
            start_gather(1 - b)

        start_scatter(b)

    drain_scatter((NCH - 1) % 2)
    plsc.subcore_barrier()

    @pl.when(sid < 10)
    def _():
        pltpu.sync_copy(acc_sh.at[pl.ds(sid * 1000, 1000)],
                        acc_out.at[pl.ds(cid * N + sid * 1000, 1000)])

    @pl.when(sid == 0)
    def _():
        pltpu.sync_copy(den_sh, den_out.at[pl.ds(cid * N, N)])


_edge_pass = functools.partial(
    pl.kernel,
    out_type=(jax.ShapeDtypeStruct((NC * N, D), jnp.float32),
              jax.ShapeDtypeStruct((NC * N,), jnp.float32)),
    mesh=plsc.VectorSubcoreMesh(core_axis_name="c", subcore_axis_name="s",
                                num_cores=NC, num_subcores=NS),
    compiler_params=pltpu.CompilerParams(needs_layout_passes=False,
                                         use_tc_tiling_on_sc=False),
    scratch_types=[
        pltpu.VMEM((N,), jnp.float32),
        pltpu.VMEM((N,), jnp.float32),
        pltpu.VMEM((2, C), jnp.int32),
        pltpu.VMEM((2, C), jnp.int32),
        pltpu.VMEM((2, C), jnp.float32),
        pltpu.VMEM((2, C, D), jnp.float32),
        pltpu.VMEM_SHARED((N, D), jnp.float32),
        pltpu.VMEM_SHARED((N,), jnp.float32),
        pltpu.SemaphoreType.DMA((2,)),
        pltpu.SemaphoreType.DMA((2,)),
        pltpu.SemaphoreType.DMA((2,)),
        pltpu.SemaphoreType.DMA((2,)),
        pltpu.SemaphoreType.DMA((2,)),
    ],
)(_edge_body)


_dense1 = pl.pallas_call(
    _dense1_body,
    out_shape=(jax.ShapeDtypeStruct((N, D), jnp.float32),
               jax.ShapeDtypeStruct((N,), jnp.float32),
               jax.ShapeDtypeStruct((N,), jnp.float32)),
)

_combine1 = pl.pallas_call(
    _combine1_body,
    out_shape=(jax.ShapeDtypeStruct((N, D), jnp.float32),
               jax.ShapeDtypeStruct((N,), jnp.float32),
               jax.ShapeDtypeStruct((N,), jnp.float32)),
)

_combine2 = pl.pallas_call(
    _combine2_body,
    out_shape=jax.ShapeDtypeStruct((N, NCLS), jnp.float32),
)


def kernel(x, edge_index, W1, a1_src, a1_dst, b1, W2, a2_src, a2_dst, b2):
    a1s = a1_src.reshape(1, D)
    a1d = a1_dst.reshape(1, D)
    h1, s1, d1 = _dense1(x, W1, a1s, a1d)

    z16 = jnp.zeros((N, D), jnp.float32)
    z1 = jnp.zeros((N,), jnp.float32)
    acc1, den1 = _edge_pass(edge_index, h1, s1, d1, z16, z1)

    W2p = jnp.zeros((D, D), jnp.float32).at[:, :NCLS].set(W2)
    a2sp = jnp.zeros((1, D), jnp.float32).at[0, :NCLS].set(a2_src)
    a2dp = jnp.zeros((1, D), jnp.float32).at[0, :NCLS].set(a2_dst)
    b2p = jnp.zeros((1, D), jnp.float32).at[0, :NCLS].set(b2)

    h2, s2, d2 = _combine1(acc1, den1, h1, a1s, a1d, b1.reshape(1, D),
                           W2p, a2sp, a2dp)

    acc2, den2 = _edge_pass(edge_index, h2, s2, d2, z16, z1)

    return _combine2(acc2, den2, h2, a2sp, a2dp, b2p)
